# baseline (speedup 1.0000x reference)
import functools

import jax
import jax.numpy as jnp
from jax import lax
from jax.experimental import pallas as pl
from jax.experimental.pallas import tpu as pltpu
from jax.experimental.pallas import tpu_sc as plsc

_N = 10000
_E = 320000
_D = 128
_H = 128
_HH = _H // 2

_NC = 2
_NS = 16
_EPS = _E // _NS
_B = 80
_NB = _EPS // _B
_CH = 624
_CHT = _N - 15 * _CH

_mesh = plsc.VectorSubcoreMesh(core_axis_name="c", subcore_axis_name="s")


def _make_sc_segment_sum(with_counts):
    out_struct = jax.ShapeDtypeStruct((_NC * _N, _HH), jnp.float32)
    out_type = [out_struct] if with_counts else out_struct
    scratch = [
        pltpu.VMEM((_NB, _B), jnp.int32),
        pltpu.VMEM((_NB, _B), jnp.int32),
        pltpu.VMEM((4, _B, _HH), jnp.float32),
        pltpu.SemaphoreType.DMA,
        pltpu.SemaphoreType.DMA,
        pltpu.SemaphoreType.DMA,
        pltpu.SemaphoreType.DMA,
        pltpu.SemaphoreType.DMA,
        pltpu.SemaphoreType.DMA,
        pltpu.SemaphoreType.DMA,
        pltpu.SemaphoreType.DMA,
        pltpu.VMEM_SHARED((_N, _HH), jnp.float32),
    ]
    if with_counts:
        out_type.append(jax.ShapeDtypeStruct((_NC * _N,), jnp.float32))
        scratch += [
            pltpu.VMEM((_B,), jnp.float32),
            pltpu.VMEM((_N,), jnp.float32),
            pltpu.VMEM_SHARED((_N,), jnp.float32),
        ]

    @functools.partial(
        pl.kernel,
        mesh=_mesh,
        compiler_params=pltpu.CompilerParams(use_tc_tiling_on_sc=False),
        out_type=out_type,
        scratch_types=scratch,
    )
    def _sc_kernel(pa_hbm, pb_hbm, src_hbm, dst_hbm, z2_hbm, z1_hbm,
                   *out_and_scratch):
        if with_counts:
            (out_hbm, cnt_hbm, srcv, dstv, ring,
             g0, g1, g2, g3, s0, s1, s2, s3,
             accsh, ones, cbuf, cntsh) = out_and_scratch
        else:
            (out_hbm, srcv, dstv, ring,
             g0, g1, g2, g3, s0, s1, s2, s3,
             accsh) = out_and_scratch
        gsem = (g0, g1, g2, g3)
        ssem = (s0, s1, s2, s3)
        cid = lax.axis_index("c")
        sid = lax.axis_index("s")

        pltpu.sync_copy(src_hbm.at[sid], srcv)
        pltpu.sync_copy(dst_hbm.at[sid], dstv)

        if with_counts:
            for i in range(_B // 16):
                ones[pl.ds(i * 16, 16)] = jnp.full((16,), 1.0,
                                                   dtype=jnp.float32)

        r0 = pl.multiple_of(sid * _CH, 8)

        @pl.when(sid < _NS - 1)
        def _():
            pltpu.sync_copy(z2_hbm.at[pl.ds(r0, _CH)],
                            accsh.at[pl.ds(r0, _CH)])

        @pl.when(sid == _NS - 1)
        def _():
            pltpu.sync_copy(z2_hbm.at[pl.ds(15 * _CH, _CHT)],
                            accsh.at[pl.ds(15 * _CH, _CHT)])

        if with_counts:
            @pl.when(sid == 0)
            def _():
                pltpu.sync_copy(z1_hbm, cbuf)
                pltpu.sync_copy(cbuf, cntsh)

        plsc.subcore_barrier()

        def _startg(j, b):
            @pl.when(cid == 0)
            def _():
                pltpu.async_copy(pa_hbm.at[srcv.at[j]], ring.at[b], gsem[b])

            @pl.when(cid == 1)
            def _():
                pltpu.async_copy(pb_hbm.at[srcv.at[j]], ring.at[b], gsem[b])

        def _waitg(b):
            pltpu.make_async_copy(pa_hbm.at[pl.ds(0, _B)], ring.at[b],
                                  gsem[b]).wait()

        def _starts(j, b, parity):
            pltpu.async_copy(ring.at[b], accsh.at[dstv.at[j]], ssem[b],
                             add=True)
            if with_counts:
                @pl.when(cid == parity)
                def _():
                    pltpu.sync_copy(ones, cntsh.at[dstv.at[j]], add=True)

        def _waits(b):
            pltpu.make_async_copy(ring.at[b], accsh.at[pl.ds(0, _B)],
                                  ssem[b]).wait()

        _startg(0, 0)
        _startg(1, 1)
        _waitg(0)
        _starts(0, 0, 0)
        _startg(2, 2)
        _waitg(1)
        _starts(1, 1, 1)
        _startg(3, 3)

        def body(k, carry):
            j = 4 * k + 2
            for o in range(4):
                b = (2 + o) % 4
                _waitg(b)
                _starts(j + o, b, o % 2)
                _waits((b + 2) % 4)
                _startg(j + o + 2, (b + 2) % 4)
            return carry

        lax.fori_loop(0, (_NB - 6) // 4, body, 0, unroll=2)

        _waitg(2)
        _starts(_NB - 4, 2, 0)
        _waits(0)
        _startg(_NB - 2, 0)
        _waitg(3)
        _starts(_NB - 3, 3, 1)
        _waits(1)
        _startg(_NB - 1, 1)
        _waitg(0)
        _starts(_NB - 2, 0, 0)
        _waits(2)
        _waitg(1)
        _starts(_NB - 1, 1, 1)
        _waits(3)
        _waits(0)
        _waits(1)

        plsc.subcore_barrier()

        o0 = pl.multiple_of(cid * _N + sid * _CH, 8)

        @pl.when(sid < _NS - 1)
        def _():
            pltpu.sync_copy(accsh.at[pl.ds(r0, _CH)],
                            out_hbm.at[pl.ds(o0, _CH)])

        @pl.when(sid == _NS - 1)
        def _():
            pltpu.sync_copy(
                accsh.at[pl.ds(15 * _CH, _CHT)],
                out_hbm.at[pl.ds(pl.multiple_of(cid * _N + 15 * _CH, 8),
                                 _CHT)])

        if with_counts:
            @pl.when(sid == 0)
            def _():
                pltpu.sync_copy(cntsh, cbuf)
                nb5 = _N // 5
                for k in range(5):
                    pltpu.sync_copy(
                        cbuf.at[pl.ds(k * nb5, nb5)],
                        cnt_hbm.at[pl.ds(
                            pl.multiple_of(k * _NC * nb5 + cid * nb5, 8),
                            nb5)])

    return _sc_kernel


_sc_segment_sum_cnt = _make_sc_segment_sum(True)
_sc_segment_sum_nocnt = _make_sc_segment_sum(False)


_BR = 2000


def _tc_stage_a_body(x_ref, w_ref, pa_ref, pb_ref, r_ref):
    o = jnp.dot(x_ref[...], w_ref[...], preferred_element_type=jnp.float32)
    pa_ref[...] = o[:, :_HH]
    pb_ref[...] = o[:, _HH:_H]
    r_ref[...] = o[:, _H:]


def _tc_stage_a(x, wcat):
    return pl.pallas_call(
        _tc_stage_a_body,
        grid=(_N // _BR,),
        in_specs=[
            pl.BlockSpec((_BR, _D), lambda i: (i, 0)),
            pl.BlockSpec((_D, 2 * _H), lambda i: (0, 0)),
        ],
        out_specs=[
            pl.BlockSpec((_BR, _HH), lambda i: (i, 0)),
            pl.BlockSpec((_BR, _HH), lambda i: (i, 0)),
            pl.BlockSpec((_BR, _H), lambda i: (i, 0)),
        ],
        out_shape=[
            jax.ShapeDtypeStruct((_N, _HH), jnp.float32),
            jax.ShapeDtypeStruct((_N, _HH), jnp.float32),
            jax.ShapeDtypeStruct((_N, _H), jnp.float32),
        ],
    )(x, wcat)


def _tc_edges_body(e_ref, src_ref, dst_ref):
    src_ref[...] = e_ref[0]
    dst_ref[...] = e_ref[1]


def _tc_edges(edge_index):
    return pl.pallas_call(
        _tc_edges_body,
        out_shape=[
            jax.ShapeDtypeStruct((_E,), jnp.int32),
            jax.ShapeDtypeStruct((_E,), jnp.int32),
        ],
    )(edge_index)


def _tc_stage_b_body(agg_ref, cnt_ref, r_ref, b_ref, w_ref,
                     p2a_ref, p2b_ref, r2_ref):
    agg = jnp.concatenate([agg_ref[0], agg_ref[1]], axis=1)
    cnt = jnp.maximum(cnt_ref[0, 0] + cnt_ref[0, 1], 1.0)
    h = jax.nn.relu(agg / cnt[:, None] + r_ref[...] + b_ref[...])
    o = jnp.dot(h, w_ref[...], preferred_element_type=jnp.float32)
    p2a_ref[...] = o[:, :_HH]
    p2b_ref[...] = o[:, _HH:_H]
    r2_ref[...] = o[:, _H:]


def _tc_stage_b(agg, cnt, r1, b1, wcat2):
    return pl.pallas_call(
        _tc_stage_b_body,
        grid=(_N // _BR,),
        in_specs=[
            pl.BlockSpec((_NC, _BR, _HH), lambda i: (0, i, 0)),
            pl.BlockSpec((1, _NC, _BR), lambda i: (i, 0, 0)),
            pl.BlockSpec((_BR, _H), lambda i: (i, 0)),
            pl.BlockSpec((1, _H), lambda i: (0, 0)),
            pl.BlockSpec((_H, 2 * _H), lambda i: (0, 0)),
        ],
        out_specs=[
            pl.BlockSpec((_BR, _HH), lambda i: (i, 0)),
            pl.BlockSpec((_BR, _HH), lambda i: (i, 0)),
            pl.BlockSpec((_BR, _H), lambda i: (i, 0)),
        ],
        out_shape=[
            jax.ShapeDtypeStruct((_N, _HH), jnp.float32),
            jax.ShapeDtypeStruct((_N, _HH), jnp.float32),
            jax.ShapeDtypeStruct((_N, _H), jnp.float32),
        ],
    )(agg, cnt, r1, b1, wcat2)


def _tc_stage_c_body(agg_ref, cnt_ref, r_ref, b_ref, wrec_ref, brec_ref,
                     z_ref, xr_ref):
    agg = jnp.concatenate([agg_ref[0], agg_ref[1]], axis=1)
    cnt = jnp.maximum(cnt_ref[0, 0] + cnt_ref[0, 1], 1.0)
    z = agg / cnt[:, None] + r_ref[...] + b_ref[...]
    z_ref[...] = z
    xr_ref[...] = jnp.dot(z, wrec_ref[...],
                          preferred_element_type=jnp.float32) + brec_ref[...]


def _tc_stage_c(agg, cnt, r2, b2, wrect, brec):
    return pl.pallas_call(
        _tc_stage_c_body,
        grid=(_N // _BR,),
        in_specs=[
            pl.BlockSpec((_NC, _BR, _HH), lambda i: (0, i, 0)),
            pl.BlockSpec((1, _NC, _BR), lambda i: (i, 0, 0)),
            pl.BlockSpec((_BR, _H), lambda i: (i, 0)),
            pl.BlockSpec((1, _H), lambda i: (0, 0)),
            pl.BlockSpec((_H, _D), lambda i: (0, 0)),
            pl.BlockSpec((1, _D), lambda i: (0, 0)),
        ],
        out_specs=[
            pl.BlockSpec((_BR, _H), lambda i: (i, 0)),
            pl.BlockSpec((_BR, _D), lambda i: (i, 0)),
        ],
        out_shape=[
            jax.ShapeDtypeStruct((_N, _H), jnp.float32),
            jax.ShapeDtypeStruct((_N, _D), jnp.float32),
        ],
    )(agg, cnt, r2, b2, wrect, brec)


def kernel(x, edge_index, Wl1, Wr1, b1, Wl2, Wr2, b2, Wrec, brec):
    z2 = jnp.zeros((_N, _HH), dtype=jnp.float32)
    z1 = jnp.zeros((_N,), dtype=jnp.float32)

    wcat1 = jnp.concatenate([Wl1.T, Wr1.T], axis=1)
    wcat2 = jnp.concatenate([Wl2.T, Wr2.T], axis=1)

    src_f, dst_f = _tc_edges(edge_index)
    p1a, p1b, r1 = _tc_stage_a(x, wcat1)
    src = src_f.reshape(_NS, _NB, _B)
    dst = dst_f.reshape(_NS, _NB, _B)
    agg1, cnt = _sc_segment_sum_cnt(p1a, p1b, src, dst, z2, z1)
    cnt2 = cnt.reshape(5, _NC, _N // 5)
    p2a, p2b, r2 = _tc_stage_b(agg1.reshape(_NC, _N, _HH), cnt2, r1,
                               b1.reshape(1, _H), wcat2)

    agg2 = _sc_segment_sum_nocnt(p2a, p2b, src, dst, z2, z1)
    z, x_recon = _tc_stage_c(agg2.reshape(_NC, _N, _HH), cnt2, r2,
                             b2.reshape(1, _H), Wrec.T, brec.reshape(1, _D))
    return (z, x_recon)


# speedup vs baseline: 1.0520x; 1.0520x over previous
import functools

import jax
import jax.numpy as jnp
from jax import lax
from jax.experimental import pallas as pl
from jax.experimental.pallas import tpu as pltpu
from jax.experimental.pallas import tpu_sc as plsc

_N = 10000
_E = 320000
_D = 128
_H = 128
_HH = _H // 2

_NC = 2
_NS = 16
_EPS = _E // _NS
_B = 80
_NB = _EPS // _B
_CH = 624
_CHT = _N - 15 * _CH

_mesh = plsc.VectorSubcoreMesh(core_axis_name="c", subcore_axis_name="s")


def _make_sc_segment_sum(with_counts):
    out_struct = jax.ShapeDtypeStruct((_NC * _N, _HH), jnp.float32)
    out_type = [out_struct] if with_counts else out_struct
    scratch = [
        pltpu.VMEM((_NB, _B), jnp.int32),
        pltpu.VMEM((_NB, _B), jnp.int32),
        pltpu.VMEM((4, _B, _HH), jnp.float32),
        pltpu.SemaphoreType.DMA,
        pltpu.SemaphoreType.DMA,
        pltpu.SemaphoreType.DMA,
        pltpu.SemaphoreType.DMA,
        pltpu.SemaphoreType.DMA,
        pltpu.SemaphoreType.DMA,
        pltpu.SemaphoreType.DMA,
        pltpu.SemaphoreType.DMA,
        pltpu.VMEM_SHARED((_N, _HH), jnp.float32),
    ]
    if with_counts:
        out_type.append(jax.ShapeDtypeStruct((_NC * _N,), jnp.float32))
        scratch += [
            pltpu.VMEM((_B,), jnp.float32),
            pltpu.VMEM((_N,), jnp.float32),
            pltpu.VMEM_SHARED((_N,), jnp.float32),
        ]

    @functools.partial(
        pl.kernel,
        mesh=_mesh,
        compiler_params=pltpu.CompilerParams(use_tc_tiling_on_sc=False),
        out_type=out_type,
        scratch_types=scratch,
    )
    def _sc_kernel(p_hbm, srca_hbm, srcb_hbm, dst_hbm, z2_hbm, z1_hbm,
                   *out_and_scratch):
        if with_counts:
            (out_hbm, cnt_hbm, srcv, dstv, ring,
             g0, g1, g2, g3, s0, s1, s2, s3,
             accsh, ones, cbuf, cntsh) = out_and_scratch
        else:
            (out_hbm, srcv, dstv, ring,
             g0, g1, g2, g3, s0, s1, s2, s3,
             accsh) = out_and_scratch
        gsem = (g0, g1, g2, g3)
        ssem = (s0, s1, s2, s3)
        cid = lax.axis_index("c")
        sid = lax.axis_index("s")

        @pl.when(cid == 0)
        def _():
            pltpu.sync_copy(srca_hbm.at[sid], srcv)

        @pl.when(cid == 1)
        def _():
            pltpu.sync_copy(srcb_hbm.at[sid], srcv)

        pltpu.sync_copy(dst_hbm.at[sid], dstv)

        if with_counts:
            for i in range(_B // 16):
                ones[pl.ds(i * 16, 16)] = jnp.full((16,), 1.0,
                                                   dtype=jnp.float32)

        r0 = pl.multiple_of(sid * _CH, 8)

        @pl.when(sid < _NS - 1)
        def _():
            pltpu.sync_copy(z2_hbm.at[pl.ds(r0, _CH)],
                            accsh.at[pl.ds(r0, _CH)])

        @pl.when(sid == _NS - 1)
        def _():
            pltpu.sync_copy(z2_hbm.at[pl.ds(15 * _CH, _CHT)],
                            accsh.at[pl.ds(15 * _CH, _CHT)])

        if with_counts:
            @pl.when(sid == 0)
            def _():
                pltpu.sync_copy(z1_hbm, cbuf)
                pltpu.sync_copy(cbuf, cntsh)

        plsc.subcore_barrier()

        def _startg(j, b):
            pltpu.async_copy(p_hbm.at[srcv.at[j]], ring.at[b], gsem[b])

        def _waitg(b):
            pltpu.make_async_copy(p_hbm.at[pl.ds(0, _B)], ring.at[b],
                                  gsem[b]).wait()

        def _starts(j, b, parity):
            pltpu.async_copy(ring.at[b], accsh.at[dstv.at[j]], ssem[b],
                             add=True)
            if with_counts:
                @pl.when(cid == parity)
                def _():
                    pltpu.sync_copy(ones, cntsh.at[dstv.at[j]], add=True)

        def _waits(b):
            pltpu.make_async_copy(ring.at[b], accsh.at[pl.ds(0, _B)],
                                  ssem[b]).wait()

        _startg(0, 0)
        _startg(1, 1)
        _waitg(0)
        _starts(0, 0, 0)
        _startg(2, 2)
        _waitg(1)
        _starts(1, 1, 1)
        _startg(3, 3)

        def body(k, carry):
            j = 4 * k + 2
            for o in range(4):
                b = (2 + o) % 4
                _waitg(b)
                _starts(j + o, b, o % 2)
                _waits((b + 2) % 4)
                _startg(j + o + 2, (b + 2) % 4)
            return carry

        lax.fori_loop(0, (_NB - 6) // 4, body, 0, unroll=False)

        _waitg(2)
        _starts(_NB - 4, 2, 0)
        _waits(0)
        _startg(_NB - 2, 0)
        _waitg(3)
        _starts(_NB - 3, 3, 1)
        _waits(1)
        _startg(_NB - 1, 1)
        _waitg(0)
        _starts(_NB - 2, 0, 0)
        _waits(2)
        _waitg(1)
        _starts(_NB - 1, 1, 1)
        _waits(3)
        _waits(0)
        _waits(1)

        plsc.subcore_barrier()

        o0 = pl.multiple_of(cid * _N + sid * _CH, 8)

        @pl.when(sid < _NS - 1)
        def _():
            pltpu.sync_copy(accsh.at[pl.ds(r0, _CH)],
                            out_hbm.at[pl.ds(o0, _CH)])

        @pl.when(sid == _NS - 1)
        def _():
            pltpu.sync_copy(
                accsh.at[pl.ds(15 * _CH, _CHT)],
                out_hbm.at[pl.ds(pl.multiple_of(cid * _N + 15 * _CH, 8),
                                 _CHT)])

        if with_counts:
            @pl.when(sid == 0)
            def _():
                pltpu.sync_copy(cntsh, cbuf)
                nb5 = _N // 5
                for k in range(5):
                    pltpu.sync_copy(
                        cbuf.at[pl.ds(k * nb5, nb5)],
                        cnt_hbm.at[pl.ds(
                            pl.multiple_of(k * _NC * nb5 + cid * nb5, 8),
                            nb5)])

    return _sc_kernel


_sc_segment_sum_cnt = _make_sc_segment_sum(True)
_sc_segment_sum_nocnt = _make_sc_segment_sum(False)


_BR = 2000


def _tc_stage_a_body(x_ref, w_ref, p_ref, r_ref):
    o = jnp.dot(x_ref[...], w_ref[...], preferred_element_type=jnp.float32)
    p_ref[...] = o[:, :_H]
    r_ref[...] = o[:, _H:]


def _tc_stage_a(x, wcat):
    return pl.pallas_call(
        _tc_stage_a_body,
        grid=(_N // _BR,),
        in_specs=[
            pl.BlockSpec((_BR, _D), lambda i: (i, 0)),
            pl.BlockSpec((_D, 2 * _H), lambda i: (0, 0)),
        ],
        out_specs=[
            pl.BlockSpec((_BR, _H), lambda i: (i, 0)),
            pl.BlockSpec((_BR, _H), lambda i: (i, 0)),
        ],
        out_shape=[
            jax.ShapeDtypeStruct((_N, _H), jnp.float32),
            jax.ShapeDtypeStruct((_N, _H), jnp.float32),
        ],
    )(x, wcat)


def _tc_edges_body(e_ref, srca_ref, srcb_ref, dst_ref):
    srca_ref[...] = e_ref[0] * 2
    srcb_ref[...] = e_ref[0] * 2 + 1
    dst_ref[...] = e_ref[1]


def _tc_edges(edge_index):
    return pl.pallas_call(
        _tc_edges_body,
        out_shape=[
            jax.ShapeDtypeStruct((_E,), jnp.int32),
            jax.ShapeDtypeStruct((_E,), jnp.int32),
            jax.ShapeDtypeStruct((_E,), jnp.int32),
        ],
    )(edge_index)


def _tc_stage_b_body(agg_ref, cnt_ref, r_ref, b_ref, w_ref,
                     p2_ref, r2_ref):
    agg = jnp.concatenate([agg_ref[0], agg_ref[1]], axis=1)
    cnt = jnp.maximum(cnt_ref[0, 0] + cnt_ref[0, 1], 1.0)
    h = jax.nn.relu(agg / cnt[:, None] + r_ref[...] + b_ref[...])
    o = jnp.dot(h, w_ref[...], preferred_element_type=jnp.float32)
    p2_ref[...] = o[:, :_H]
    r2_ref[...] = o[:, _H:]


def _tc_stage_b(agg, cnt, r1, b1, wcat2):
    return pl.pallas_call(
        _tc_stage_b_body,
        grid=(_N // _BR,),
        in_specs=[
            pl.BlockSpec((_NC, _BR, _HH), lambda i: (0, i, 0)),
            pl.BlockSpec((1, _NC, _BR), lambda i: (i, 0, 0)),
            pl.BlockSpec((_BR, _H), lambda i: (i, 0)),
            pl.BlockSpec((1, _H), lambda i: (0, 0)),
            pl.BlockSpec((_H, 2 * _H), lambda i: (0, 0)),
        ],
        out_specs=[
            pl.BlockSpec((_BR, _H), lambda i: (i, 0)),
            pl.BlockSpec((_BR, _H), lambda i: (i, 0)),
        ],
        out_shape=[
            jax.ShapeDtypeStruct((_N, _H), jnp.float32),
            jax.ShapeDtypeStruct((_N, _H), jnp.float32),
        ],
    )(agg, cnt, r1, b1, wcat2)


def _tc_stage_c_body(agg_ref, cnt_ref, r_ref, b_ref, wrec_ref, brec_ref,
                     z_ref, xr_ref):
    agg = jnp.concatenate([agg_ref[0], agg_ref[1]], axis=1)
    cnt = jnp.maximum(cnt_ref[0, 0] + cnt_ref[0, 1], 1.0)
    z = agg / cnt[:, None] + r_ref[...] + b_ref[...]
    z_ref[...] = z
    xr_ref[...] = jnp.dot(z, wrec_ref[...],
                          preferred_element_type=jnp.float32) + brec_ref[...]


def _tc_stage_c(agg, cnt, r2, b2, wrect, brec):
    return pl.pallas_call(
        _tc_stage_c_body,
        grid=(_N // _BR,),
        in_specs=[
            pl.BlockSpec((_NC, _BR, _HH), lambda i: (0, i, 0)),
            pl.BlockSpec((1, _NC, _BR), lambda i: (i, 0, 0)),
            pl.BlockSpec((_BR, _H), lambda i: (i, 0)),
            pl.BlockSpec((1, _H), lambda i: (0, 0)),
            pl.BlockSpec((_H, _D), lambda i: (0, 0)),
            pl.BlockSpec((1, _D), lambda i: (0, 0)),
        ],
        out_specs=[
            pl.BlockSpec((_BR, _H), lambda i: (i, 0)),
            pl.BlockSpec((_BR, _D), lambda i: (i, 0)),
        ],
        out_shape=[
            jax.ShapeDtypeStruct((_N, _H), jnp.float32),
            jax.ShapeDtypeStruct((_N, _D), jnp.float32),
        ],
    )(agg, cnt, r2, b2, wrect, brec)


def kernel(x, edge_index, Wl1, Wr1, b1, Wl2, Wr2, b2, Wrec, brec):
    z2 = jnp.zeros((_N, _HH), dtype=jnp.float32)
    z1 = jnp.zeros((_N,), dtype=jnp.float32)

    wcat1 = jnp.concatenate([Wl1.T, Wr1.T], axis=1)
    wcat2 = jnp.concatenate([Wl2.T, Wr2.T], axis=1)

    srca_f, srcb_f, dst_f = _tc_edges(edge_index)
    p1, r1 = _tc_stage_a(x, wcat1)
    srca = srca_f.reshape(_NS, _NB, _B)
    srcb = srcb_f.reshape(_NS, _NB, _B)
    dst = dst_f.reshape(_NS, _NB, _B)
    agg1, cnt = _sc_segment_sum_cnt(p1.reshape(_NC * _N, _HH),
                                    srca, srcb, dst, z2, z1)
    cnt2 = cnt.reshape(5, _NC, _N // 5)
    p2, r2 = _tc_stage_b(agg1.reshape(_NC, _N, _HH), cnt2, r1,
                         b1.reshape(1, _H), wcat2)

    agg2 = _sc_segment_sum_nocnt(p2.reshape(_NC * _N, _HH),
                                 srca, srcb, dst, z2, z1)
    z, x_recon = _tc_stage_c(agg2.reshape(_NC, _N, _HH), cnt2, r2,
                             b2.reshape(1, _H), Wrec.T, brec.reshape(1, _D))
    return (z, x_recon)
